# baseline (speedup 1.0000x reference)
import functools

import jax
import jax.numpy as jnp
from jax import lax
from jax.experimental import pallas as pl
from jax.experimental.pallas import tpu as pltpu
from jax.experimental.pallas import tpu_sc as plsc

B = 16384
DIM = 32
RAW_DIM = 96
OUT_DIM = 64
NC = 2
NS = 16
NW = NC * NS
BPW = B // NW


def _sc_gather3(category, brand, store, t_cat, t_brand, t_store):
    mesh = plsc.VectorSubcoreMesh(core_axis_name="c", subcore_axis_name="s")
    out_t = jax.ShapeDtypeStruct((B, DIM), jnp.float32)

    @functools.partial(
        pl.kernel,
        mesh=mesh,
        out_type=[out_t, out_t, out_t],
        compiler_params=pltpu.CompilerParams(use_tc_tiling_on_sc=False),
        scratch_types=[
            pltpu.VMEM((BPW,), jnp.int32),
            pltpu.VMEM((BPW, DIM), jnp.float32),
            pltpu.SemaphoreType.DMA,
        ],
    )
    def k(ci, bi, si, tc_, tb_, ts_, oc, ob, osr, idx_v, rows_v, sem):
        wid = lax.axis_index("s") * NC + lax.axis_index("c")
        base = wid * BPW
        for i_hbm, t_hbm, o_hbm in ((ci, tc_, oc), (bi, tb_, ob), (si, ts_, osr)):
            pltpu.sync_copy(i_hbm.at[pl.ds(base, BPW)], idx_v)
            pltpu.async_copy(t_hbm.at[idx_v], rows_v, sem).wait()
            pltpu.sync_copy(rows_v, o_hbm.at[pl.ds(base, BPW)])

    return k(category, brand, store, t_cat, t_brand, t_store)


BM = 2048


def _tc_project(e_cat, e_brand, e_store, Wt, b2):
    def body(a_ref, b_ref, c_ref, w_ref, bias_ref, o_ref):
        x = jnp.concatenate([a_ref[...], b_ref[...], c_ref[...]], axis=1)
        y = jnp.dot(x, w_ref[...], preferred_element_type=jnp.float32)
        o_ref[...] = jnp.maximum(y + bias_ref[...], 0.0)

    return pl.pallas_call(
        body,
        grid=(B // BM,),
        in_specs=[
            pl.BlockSpec((BM, DIM), lambda i: (i, 0)),
            pl.BlockSpec((BM, DIM), lambda i: (i, 0)),
            pl.BlockSpec((BM, DIM), lambda i: (i, 0)),
            pl.BlockSpec((RAW_DIM, OUT_DIM), lambda i: (0, 0)),
            pl.BlockSpec((1, OUT_DIM), lambda i: (0, 0)),
        ],
        out_specs=pl.BlockSpec((BM, OUT_DIM), lambda i: (i, 0)),
        out_shape=jax.ShapeDtypeStruct((B, OUT_DIM), jnp.float32),
    )(e_cat, e_brand, e_store, Wt, b2)


def kernel(category, brand, store, emb_category, emb_brand, emb_store, W, b):
    ci = category.astype(jnp.int32)
    bi = brand.astype(jnp.int32)
    si = store.astype(jnp.int32)
    e_cat, e_brand, e_store = _sc_gather3(
        ci, bi, si, emb_category, emb_brand, emb_store
    )
    Wt = W.T
    b2 = b.reshape(1, OUT_DIM)
    return _tc_project(e_cat, e_brand, e_store, Wt, b2)
